# Initial kernel scaffold; baseline (speedup 1.0000x reference)
import functools

import jax
import jax.numpy as jnp
from jax import lax
from jax.experimental import pallas as pl
from jax.experimental.pallas import tpu as pltpu
from jax.experimental.pallas import tpu_sc as plsc

D = 32
B = 16384
L = 200
NC = 2
NS = 16
LANES = 16
NW = NC * NS
TOK = B * L
GW = 128
ROWS = TOK // GW
ROWS_PER_TILE = ROWS // NW
PANEL = 80
NPANEL = ROWS_PER_TILE // PANEL
B_PER_SC = B // NC
B_PER_TILE = B // NW
SCALE = 1.0 / L


def _embed_body(tokens_hbm, seg_hbm, table_hbm, out_hbm,
                idx_v, seg_v, buf0, buf1, outbuf, acc, sem0, sem1):
    c = lax.axis_index("c")
    s = lax.axis_index("s")
    tile = c * NS + s
    tok_row0 = tile * ROWS_PER_TILE
    acc_row0 = s * B_PER_TILE
    out_row0 = c * B_PER_SC + s * B_PER_TILE

    zero = jnp.zeros((LANES,), jnp.float32)

    @pl.loop(0, GW)
    def _(i):
        buf0[i, pl.ds(0, LANES)] = zero
        buf0[i, pl.ds(LANES, LANES)] = zero

    for q in range(B_PER_TILE // GW):
        pltpu.sync_copy(buf0, acc.at[pl.ds(acc_row0 + q * GW, GW)])

    def start_gather(j, buf, sem):
        return pltpu.async_copy(table_hbm.at[idx_v.at[j]], buf, sem)

    def wait_gather(j, buf, sem):
        pltpu.make_async_copy(table_hbm.at[idx_v.at[j]], buf, sem).wait()

    def scatter_add(j, buf):
        pltpu.sync_copy(buf, acc.at[seg_v.at[j]], add=True)

    @pl.loop(0, NPANEL)
    def _(p):
        r0 = tok_row0 + p * PANEL
        pltpu.sync_copy(tokens_hbm.at[pl.ds(r0, PANEL)], idx_v)
        pltpu.sync_copy(seg_hbm.at[pl.ds(r0, PANEL)], seg_v)
        start_gather(0, buf0, sem0)

        @pl.loop(0, PANEL, step=2)
        def _(j):
            start_gather(j + 1, buf1, sem1)
            wait_gather(j, buf0, sem0)
            scatter_add(j, buf0)

            @pl.when(j + 2 < PANEL)
            def _():
                start_gather(j + 2, buf0, sem0)

            wait_gather(j + 1, buf1, sem1)
            scatter_add(j + 1, buf1)

    pltpu.sync_copy(acc.at[pl.ds(acc_row0, B_PER_TILE)], outbuf)

    @pl.loop(0, B_PER_TILE)
    def _(i):
        for k in range(D // LANES):
            sl = pl.ds(k * LANES, LANES)
            outbuf[i, sl] = outbuf[i, sl] * SCALE

    pltpu.sync_copy(outbuf, out_hbm.at[pl.ds(out_row0, B_PER_TILE)])


@jax.jit
def kernel(tokens, table):
    tokens2d = tokens.astype(jnp.int32).reshape(ROWS, GW)
    b_local = (jnp.arange(B, dtype=jnp.int32) % B_PER_SC)
    seg2d = jnp.broadcast_to(b_local[:, None], (B, L)).reshape(ROWS, GW)

    mesh = plsc.VectorSubcoreMesh(core_axis_name="c", subcore_axis_name="s")
    run = pl.kernel(
        _embed_body,
        out_type=jax.ShapeDtypeStruct((B, D), jnp.float32),
        mesh=mesh,
        scratch_types=[
            pltpu.VMEM((PANEL, GW), jnp.int32),
            pltpu.VMEM((PANEL, GW), jnp.int32),
            pltpu.VMEM((GW, D), jnp.float32),
            pltpu.VMEM((GW, D), jnp.float32),
            pltpu.VMEM((B_PER_TILE, D), jnp.float32),
            pltpu.VMEM_SHARED((B_PER_SC, D), jnp.float32),
            pltpu.SemaphoreType.DMA,
            pltpu.SemaphoreType.DMA,
        ],
    )
    return run(tokens2d, seg2d, table)


# speedup vs baseline: 11.8301x; 11.8301x over previous
import functools

import jax
import jax.numpy as jnp
from jax import lax
from jax.experimental import pallas as pl
from jax.experimental.pallas import tpu as pltpu
from jax.experimental.pallas import tpu_sc as plsc

D = 32
B = 16384
L = 200
NC = 2
NS = 16
LANES = 16
NW = NC * NS
TOK = B * L
GW = 128
ROWS = TOK // GW
ROWS_PER_TILE = ROWS // NW
PANEL = 80
NPANEL = ROWS_PER_TILE // PANEL
B_PER_SC = B // NC
B_PER_TILE = B // NW
SCALE = 1.0 / L


def _embed_body(tokens_hbm, seg_hbm, table_hbm, out_hbm,
                idx_v, seg_v, buf0, buf1, outbuf, acc, sem0, sem1):
    c = lax.axis_index("c")
    s = lax.axis_index("s")
    tile = c * NS + s
    tok_row0 = tile * ROWS_PER_TILE
    acc_row0 = s * B_PER_TILE
    out_row0 = c * B_PER_SC + s * B_PER_TILE

    zero = jnp.zeros((LANES,), jnp.float32)

    @pl.loop(0, GW)
    def _(i):
        buf0[i, pl.ds(0, LANES)] = zero
        buf0[i, pl.ds(LANES, LANES)] = zero

    for q in range(B_PER_TILE // GW):
        pltpu.sync_copy(buf0, acc.at[pl.ds(acc_row0 + q * GW, GW)])

    def start_gather(j, buf, sem):
        return pltpu.async_copy(table_hbm.at[idx_v.at[j]], buf, sem)

    def wait_gather(j, buf, sem):
        pltpu.make_async_copy(table_hbm.at[idx_v.at[j]], buf, sem).wait()

    def scatter_add(j, buf):
        pltpu.sync_copy(buf, acc.at[seg_v.at[j]], add=True)

    @pl.loop(0, NPANEL)
    def _(p):
        r0 = tok_row0 + p * PANEL
        pltpu.sync_copy(tokens_hbm.at[pl.ds(r0, PANEL)], idx_v)
        pltpu.sync_copy(seg_hbm.at[pl.ds(r0, PANEL)], seg_v)
        start_gather(0, buf0, sem0)

        @pl.loop(0, PANEL, step=2)
        def _(j):
            start_gather(j + 1, buf1, sem1)
            wait_gather(j, buf0, sem0)
            scatter_add(j, buf0)

            @pl.when(j + 2 < PANEL)
            def _():
                start_gather(j + 2, buf0, sem0)

            wait_gather(j + 1, buf1, sem1)
            scatter_add(j + 1, buf1)

    pltpu.sync_copy(acc.at[pl.ds(acc_row0, B_PER_TILE)], outbuf)

    @pl.loop(0, B_PER_TILE)
    def _(i):
        for k in range(D // LANES):
            sl = pl.ds(k * LANES, LANES)
            outbuf[i, sl] = outbuf[i, sl] * SCALE

    pltpu.sync_copy(outbuf, out_hbm.at[pl.ds(out_row0, B_PER_TILE)])


@jax.jit
def kernel(tokens, table):
    tokens2d = tokens.astype(jnp.int32).reshape(ROWS, GW)
    b_local = (jnp.arange(B, dtype=jnp.int32) % B_PER_SC)
    seg2d = jnp.broadcast_to(b_local[:, None], (B, L)).reshape(ROWS, GW)

    mesh = plsc.VectorSubcoreMesh(core_axis_name="c", subcore_axis_name="s")
    run = pl.kernel(
        _embed_body,
        out_type=jax.ShapeDtypeStruct((B, D), jnp.float32),
        mesh=mesh,
        compiler_params=pltpu.CompilerParams(use_tc_tiling_on_sc=False),
        scratch_types=[
            pltpu.VMEM((PANEL, GW), jnp.int32),
            pltpu.VMEM((PANEL, GW), jnp.int32),
            pltpu.VMEM((GW, D), jnp.float32),
            pltpu.VMEM((GW, D), jnp.float32),
            pltpu.VMEM((B_PER_TILE, D), jnp.float32),
            pltpu.VMEM_SHARED((B_PER_SC, D), jnp.float32),
            pltpu.SemaphoreType.DMA,
            pltpu.SemaphoreType.DMA,
        ],
    )
    return run(tokens2d, seg2d, table)


# speedup vs baseline: 12.8549x; 1.0866x over previous
import functools

import jax
import jax.numpy as jnp
from jax import lax
from jax.experimental import pallas as pl
from jax.experimental.pallas import tpu as pltpu
from jax.experimental.pallas import tpu_sc as plsc

D = 32
B = 16384
L = 200
NC = 2
NS = 16
LANES = 16
NW = NC * NS
TOK = B * L
GW = 128
ROWS = TOK // GW
ROWS_PER_TILE = ROWS // NW
PANEL = 80
NPANEL = ROWS_PER_TILE // PANEL
GB = 8
NG = PANEL // GB
SW = GB * GW
SROWS = TOK // SW
SROWS_PER_TILE = SROWS // NW
B_PER_SC = B // NC
B_PER_TILE = B // NW
SCALE = 1.0 / L


def _embed_body(tokens_hbm, seg_hbm, table_hbm, out_hbm,
                idx_v, seg_v, buf0, buf1, outbuf, acc, sem0, sem1):
    c = lax.axis_index("c")
    s = lax.axis_index("s")
    tile = c * NS + s
    tok_row0 = tile * ROWS_PER_TILE
    acc_row0 = s * B_PER_TILE
    out_row0 = c * B_PER_SC + s * B_PER_TILE

    zero = jnp.zeros((LANES,), jnp.float32)

    @pl.loop(0, B_PER_TILE)
    def _(i):
        outbuf[i, pl.ds(0, LANES)] = zero
        outbuf[i, pl.ds(LANES, LANES)] = zero

    pltpu.sync_copy(outbuf, acc.at[pl.ds(acc_row0, B_PER_TILE)])

    def start_gather(g, buf, sem):
        return pltpu.async_copy(table_hbm.at[idx_v.at[g]], buf, sem)

    def wait_gather(buf, sem):
        pltpu.make_async_copy(table_hbm.at[idx_v.at[0]], buf, sem).wait()

    def scatter_add(g, buf):
        pltpu.sync_copy(buf, acc.at[seg_v.at[g]], add=True)

    srow0 = tile * SROWS_PER_TILE

    @pl.loop(0, NPANEL)
    def _(p):
        r0 = srow0 + p * NG
        pltpu.sync_copy(tokens_hbm.at[pl.ds(r0, NG)], idx_v)
        pltpu.sync_copy(seg_hbm.at[pl.ds(r0, NG)], seg_v)
        start_gather(0, buf0, sem0)

        @pl.loop(0, NG, step=2)
        def _(g):
            start_gather(g + 1, buf1, sem1)
            wait_gather(buf0, sem0)
            scatter_add(g, buf0)

            @pl.when(g + 2 < NG)
            def _():
                start_gather(g + 2, buf0, sem0)

            wait_gather(buf1, sem1)
            scatter_add(g + 1, buf1)

    pltpu.sync_copy(acc.at[pl.ds(acc_row0, B_PER_TILE)], outbuf)

    @pl.loop(0, B_PER_TILE)
    def _(i):
        for k in range(D // LANES):
            sl = pl.ds(k * LANES, LANES)
            outbuf[i, sl] = outbuf[i, sl] * SCALE

    pltpu.sync_copy(outbuf, out_hbm.at[pl.ds(out_row0, B_PER_TILE)])


@jax.jit
def kernel(tokens, table):
    tokens2d = tokens.astype(jnp.int32).reshape(SROWS, SW)
    b_local = (jnp.arange(B, dtype=jnp.int32) % B_PER_SC)
    seg2d = jnp.broadcast_to(b_local[:, None], (B, L)).reshape(SROWS, SW)

    mesh = plsc.VectorSubcoreMesh(core_axis_name="c", subcore_axis_name="s")
    run = pl.kernel(
        _embed_body,
        out_type=jax.ShapeDtypeStruct((B, D), jnp.float32),
        mesh=mesh,
        compiler_params=pltpu.CompilerParams(use_tc_tiling_on_sc=False),
        scratch_types=[
            pltpu.VMEM((NG, SW), jnp.int32),
            pltpu.VMEM((NG, SW), jnp.int32),
            pltpu.VMEM((SW, D), jnp.float32),
            pltpu.VMEM((SW, D), jnp.float32),
            pltpu.VMEM((B_PER_TILE, D), jnp.float32),
            pltpu.VMEM_SHARED((B_PER_SC, D), jnp.float32),
            pltpu.SemaphoreType.DMA,
            pltpu.SemaphoreType.DMA,
        ],
    )
    return run(tokens2d, seg2d, table)


# speedup vs baseline: 15.9840x; 1.2434x over previous
import jax
import jax.numpy as jnp
from jax import lax
from jax.experimental import pallas as pl
from jax.experimental.pallas import tpu as pltpu
from jax.experimental.pallas import tpu_sc as plsc

D = 32
B = 16384
L = 200
NC = 2
NS = 16
LANES = 16
NW = NC * NS
TOK = B * L
TOK_PER_TILE = TOK // NW
B_PER_TILE = B // NW
SW = 4 * L
SPP = 16
PANELTOK = SW * SPP
NPANEL = TOK_PER_TILE // PANELTOK
ROWS_PER_PANEL = PANELTOK // L
UNROLL = 8
SCALE = 1.0 / L


def _embed_body(tokens_hbm, table_hbm, out_hbm,
                idx_v, buf0, buf1, outbuf, sem0, sem1):
    c = lax.axis_index("c")
    s = lax.axis_index("s")
    tile = c * NS + s
    tok0 = tile * TOK_PER_TILE
    out_row0 = tile * B_PER_TILE

    vzero = jnp.zeros((LANES,), jnp.float32)

    def start_gather(g, buf, sem):
        pltpu.async_copy(table_hbm.at[idx_v.at[pl.ds(g * SW, SW)]], buf, sem)

    def wait_gather(buf, sem):
        pltpu.make_async_copy(table_hbm.at[idx_v.at[pl.ds(0, SW)]], buf, sem).wait()

    def accumulate(buf, row0):
        for q in range(SW // L):
            def body(i, carry):
                a0, a1 = carry
                for u in range(UNROLL):
                    r = q * L + i * UNROLL + u
                    a0 = a0 + buf[r, pl.ds(0, LANES)]
                    a1 = a1 + buf[r, pl.ds(LANES, LANES)]
                return (a0, a1)

            a0, a1 = lax.fori_loop(0, L // UNROLL, body, (vzero, vzero))
            outbuf[row0 + q, pl.ds(0, LANES)] = a0 * SCALE
            outbuf[row0 + q, pl.ds(LANES, LANES)] = a1 * SCALE

    @pl.loop(0, NPANEL)
    def _(p):
        pltpu.sync_copy(tokens_hbm.at[pl.ds(tok0 + p * PANELTOK, PANELTOK)], idx_v)
        start_gather(0, buf0, sem0)

        @pl.loop(0, SPP, step=2)
        def _(g):
            start_gather(g + 1, buf1, sem1)
            wait_gather(buf0, sem0)
            accumulate(buf0, p * ROWS_PER_PANEL + g * (SW // L))

            @pl.when(g + 2 < SPP)
            def _():
                start_gather(g + 2, buf0, sem0)

            wait_gather(buf1, sem1)
            accumulate(buf1, p * ROWS_PER_PANEL + (g + 1) * (SW // L))

    pltpu.sync_copy(outbuf, out_hbm.at[pl.ds(out_row0, B_PER_TILE)])


@jax.jit
def kernel(tokens, table):
    tokens1d = tokens.astype(jnp.int32).reshape(TOK)

    mesh = plsc.VectorSubcoreMesh(core_axis_name="c", subcore_axis_name="s")
    run = pl.kernel(
        _embed_body,
        out_type=jax.ShapeDtypeStruct((B, D), jnp.float32),
        mesh=mesh,
        compiler_params=pltpu.CompilerParams(use_tc_tiling_on_sc=False),
        scratch_types=[
            pltpu.VMEM((PANELTOK,), jnp.int32),
            pltpu.VMEM((SW, D), jnp.float32),
            pltpu.VMEM((SW, D), jnp.float32),
            pltpu.VMEM((B_PER_TILE, D), jnp.float32),
            pltpu.SemaphoreType.DMA,
            pltpu.SemaphoreType.DMA,
        ],
    )
    return run(tokens1d, table)
